# initial kernel scaffold (unmeasured)
import jax
import jax.numpy as jnp
from jax import lax
from jax.experimental import pallas as pl
from jax.experimental.pallas import tpu as pltpu

N_DEV = 4
M = 4096
K_SHARD = 1024
N_TOT = 2048
MC = M // N_DEV
NH = N_TOT // 2


def kernel(x, w_mat, scale_x, scale_w):
    scale = (scale_x[0] * scale_w[0]).reshape(1, 1).astype(jnp.float32)

    def body(x_ref, w_ref, s_ref, out_ref, comm0, comm1,
             rs_send0, rs_recv0, rs_send1, rs_recv1,
             ag_send0, ag_recv0, ag_send1, ag_recv1, credit_sem):
        i = lax.axis_index("i")
        right = lax.rem(i + 1, N_DEV)
        left = lax.rem(i + N_DEV - 1, N_DEV)
        tgt = (right, left)

        barrier = pltpu.get_barrier_semaphore()
        for nbr in (left, right):
            pl.semaphore_signal(barrier, inc=1, device_id=(nbr,),
                                device_id_type=pl.DeviceIdType.MESH)
        pl.semaphore_wait(barrier, 2)

        def mod4(v):
            return lax.rem(v + 2 * N_DEV, N_DEV)

        def pgemm(c, d):
            xa = x_ref[pl.ds(c * MC, MC), :].astype(jnp.bfloat16)
            wb = w_ref[:, d * NH:(d + 1) * NH].astype(jnp.bfloat16)
            return jnp.dot(xa, wb, preferred_element_type=jnp.float32)

        comms = (comm0, comm1)
        rs_send = (rs_send0, rs_send1)
        rs_recv = (rs_recv0, rs_recv1)

        comm0[0, :, :] = pgemm(mod4(i - 1), 0)
        comm1[0, :, :] = pgemm(mod4(i + 1), 1)

        for h in range(N_DEV - 1):
            send_slot = h % 2
            recv_slot = (h + 1) % 2
            if h == 2:
                pl.semaphore_wait(credit_sem, 2)
            rdmas = []
            for d in range(2):
                r = pltpu.make_async_remote_copy(
                    src_ref=comms[d].at[send_slot],
                    dst_ref=comms[d].at[recv_slot],
                    send_sem=rs_send[d].at[h],
                    recv_sem=rs_recv[d].at[h],
                    device_id=(tgt[d],),
                    device_id_type=pl.DeviceIdType.MESH,
                )
                r.start()
                rdmas.append(r)
            for r in rdmas:
                r.wait()
            if h == 1:
                pl.semaphore_signal(credit_sem, inc=1, device_id=(left,),
                                    device_id_type=pl.DeviceIdType.MESH)
                pl.semaphore_signal(credit_sem, inc=1, device_id=(right,),
                                    device_id_type=pl.DeviceIdType.MESH)
            if h < N_DEV - 2:
                comm0[recv_slot, :, :] = (
                    comm0[recv_slot, :, :] + pgemm(mod4(i - 2 - h), 0))
                comm1[recv_slot, :, :] = (
                    comm1[recv_slot, :, :] + pgemm(mod4(i + 2 + h), 1))
            else:
                s = s_ref[0, 0]
                for d in range(2):
                    acc = comms[d][recv_slot, :, :] + pgemm(i, d)
                    y = acc * s
                    z = jnp.clip(y, -60.0, 60.0)
                    res = y / (1.0 + jnp.exp(-z))
                    out_ref[pl.ds(i * MC, MC), d * NH:(d + 1) * NH] = res

        ag_send = (ag_send0, ag_send1)
        ag_recv = (ag_recv0, ag_recv1)
        for h in range(N_DEV - 1):
            send_c = (mod4(i - h), mod4(i + h))
            recv_c = (mod4(i - 1 - h), mod4(i + 1 + h))
            waits = []
            for d in range(2):
                send_desc = pltpu.make_async_remote_copy(
                    src_ref=out_ref.at[pl.ds(send_c[d] * MC, MC),
                                       pl.ds(d * NH, NH)],
                    dst_ref=out_ref.at[pl.ds(send_c[d] * MC, MC),
                                       pl.ds(d * NH, NH)],
                    send_sem=ag_send[d].at[h],
                    recv_sem=ag_recv[d].at[h],
                    device_id=(tgt[d],),
                    device_id_type=pl.DeviceIdType.MESH,
                )
                send_desc.start()
                recv_desc = pltpu.make_async_remote_copy(
                    src_ref=out_ref.at[pl.ds(recv_c[d] * MC, MC),
                                       pl.ds(d * NH, NH)],
                    dst_ref=out_ref.at[pl.ds(recv_c[d] * MC, MC),
                                       pl.ds(d * NH, NH)],
                    send_sem=ag_send[d].at[h],
                    recv_sem=ag_recv[d].at[h],
                    device_id=(tgt[d],),
                    device_id_type=pl.DeviceIdType.MESH,
                )
                waits.append((send_desc, recv_desc))
            for send_desc, recv_desc in waits:
                send_desc.wait_send()
                recv_desc.wait_recv()

    return pl.pallas_call(
        body,
        out_shape=jax.ShapeDtypeStruct((M, N_TOT), jnp.float32),
        in_specs=[
            pl.BlockSpec(memory_space=pltpu.VMEM),
            pl.BlockSpec(memory_space=pltpu.VMEM),
            pl.BlockSpec(memory_space=pltpu.SMEM),
        ],
        out_specs=pl.BlockSpec(memory_space=pltpu.VMEM),
        scratch_shapes=[
            pltpu.VMEM((2, MC, NH), jnp.float32),
            pltpu.VMEM((2, MC, NH), jnp.float32),
            pltpu.SemaphoreType.DMA((N_DEV - 1,)),
            pltpu.SemaphoreType.DMA((N_DEV - 1,)),
            pltpu.SemaphoreType.DMA((N_DEV - 1,)),
            pltpu.SemaphoreType.DMA((N_DEV - 1,)),
            pltpu.SemaphoreType.DMA((N_DEV - 1,)),
            pltpu.SemaphoreType.DMA((N_DEV - 1,)),
            pltpu.SemaphoreType.DMA((N_DEV - 1,)),
            pltpu.SemaphoreType.DMA((N_DEV - 1,)),
            pltpu.SemaphoreType.REGULAR,
        ],
        compiler_params=pltpu.CompilerParams(collective_id=0),
    )(x, w_mat, scale)


# baseline (device time: 332314 ns/iter reference)
import jax
import jax.numpy as jnp
from jax import lax
from jax.experimental import pallas as pl
from jax.experimental.pallas import tpu as pltpu

N_DEV = 4
M = 4096
K_SHARD = 1024
N_TOT = 2048
MC = M // N_DEV
NH = N_TOT // 2


def kernel(x, w_mat, scale_x, scale_w):
    scale = (scale_x[0] * scale_w[0]).reshape(1, 1).astype(jnp.float32)

    def body(x_ref, w_ref, s_ref, out_ref, comm0, comm1, epi,
             rs_send0, rs_recv0, rs_send1, rs_recv1,
             ag_send0, ag_recv0, ag_send1, ag_recv1,
             epi_sems, credit_sem):
        i = lax.axis_index("i")
        right = lax.rem(i + 1, N_DEV)
        left = lax.rem(i + N_DEV - 1, N_DEV)
        tgt = (right, left)

        barrier = pltpu.get_barrier_semaphore()
        for nbr in (left, right):
            pl.semaphore_signal(barrier, inc=1, device_id=(nbr,),
                                device_id_type=pl.DeviceIdType.MESH)
        pl.semaphore_wait(barrier, 2)

        def mod4(v):
            return lax.rem(v + 2 * N_DEV, N_DEV)

        def pgemm(c, d):
            xa = x_ref[pl.ds(c * MC, MC), :].astype(jnp.bfloat16)
            wb = w_ref[:, d * NH:(d + 1) * NH].astype(jnp.bfloat16)
            return jnp.dot(xa, wb, preferred_element_type=jnp.float32)

        comms = (comm0, comm1)
        rs_send = (rs_send0, rs_send1)
        rs_recv = (rs_recv0, rs_recv1)

        comm0[0, :, :] = pgemm(mod4(i - 1), 0)
        comm1[0, :, :] = pgemm(mod4(i + 1), 1)

        for h in range(N_DEV - 1):
            send_slot = h % 2
            recv_slot = (h + 1) % 2
            if h == 2:
                pl.semaphore_wait(credit_sem, 2)
            rdmas = []
            for d in range(2):
                r = pltpu.make_async_remote_copy(
                    src_ref=comms[d].at[send_slot],
                    dst_ref=comms[d].at[recv_slot],
                    send_sem=rs_send[d].at[h],
                    recv_sem=rs_recv[d].at[h],
                    device_id=(tgt[d],),
                    device_id_type=pl.DeviceIdType.MESH,
                )
                r.start()
                rdmas.append(r)
            for r in rdmas:
                r.wait()
            if h == 1:
                pl.semaphore_signal(credit_sem, inc=1, device_id=(left,),
                                    device_id_type=pl.DeviceIdType.MESH)
                pl.semaphore_signal(credit_sem, inc=1, device_id=(right,),
                                    device_id_type=pl.DeviceIdType.MESH)
            if h < N_DEV - 2:
                comm0[recv_slot, :, :] = (
                    comm0[recv_slot, :, :] + pgemm(mod4(i - 2 - h), 0))
                comm1[recv_slot, :, :] = (
                    comm1[recv_slot, :, :] + pgemm(mod4(i + 2 + h), 1))
            else:
                s = s_ref[0, 0]
                copies = []
                for d in range(2):
                    y = (comms[d][recv_slot, :, :] + pgemm(i, d)) * s
                    z = jnp.clip(y, -60.0, 60.0)
                    epi[d, :, :] = y / (1.0 + jnp.exp(-z))
                    cp = pltpu.make_async_copy(
                        epi.at[d],
                        out_ref.at[pl.ds(i * MC, MC), pl.ds(d * NH, NH)],
                        epi_sems.at[d],
                    )
                    cp.start()
                    copies.append(cp)
                for cp in copies:
                    cp.wait()

        ag_send = (ag_send0, ag_send1)
        ag_recv = (ag_recv0, ag_recv1)
        for h in range(N_DEV - 1):
            send_c = (mod4(i - h), mod4(i + h))
            recv_c = (mod4(i - 1 - h), mod4(i + 1 + h))
            waits = []
            for d in range(2):
                send_desc = pltpu.make_async_remote_copy(
                    src_ref=out_ref.at[pl.ds(send_c[d] * MC, MC),
                                       pl.ds(d * NH, NH)],
                    dst_ref=out_ref.at[pl.ds(send_c[d] * MC, MC),
                                       pl.ds(d * NH, NH)],
                    send_sem=ag_send[d].at[h],
                    recv_sem=ag_recv[d].at[h],
                    device_id=(tgt[d],),
                    device_id_type=pl.DeviceIdType.MESH,
                )
                send_desc.start()
                recv_desc = pltpu.make_async_remote_copy(
                    src_ref=out_ref.at[pl.ds(recv_c[d] * MC, MC),
                                       pl.ds(d * NH, NH)],
                    dst_ref=out_ref.at[pl.ds(recv_c[d] * MC, MC),
                                       pl.ds(d * NH, NH)],
                    send_sem=ag_send[d].at[h],
                    recv_sem=ag_recv[d].at[h],
                    device_id=(tgt[d],),
                    device_id_type=pl.DeviceIdType.MESH,
                )
                waits.append((send_desc, recv_desc))
            for send_desc, recv_desc in waits:
                send_desc.wait_send()
                recv_desc.wait_recv()

    return pl.pallas_call(
        body,
        out_shape=jax.ShapeDtypeStruct((M, N_TOT), jnp.float32),
        in_specs=[
            pl.BlockSpec(memory_space=pltpu.VMEM),
            pl.BlockSpec(memory_space=pltpu.VMEM),
            pl.BlockSpec(memory_space=pltpu.SMEM),
        ],
        out_specs=pl.BlockSpec(memory_space=pltpu.MemorySpace.HBM),
        scratch_shapes=[
            pltpu.VMEM((2, MC, NH), jnp.float32),
            pltpu.VMEM((2, MC, NH), jnp.float32),
            pltpu.VMEM((2, MC, NH), jnp.float32),
            pltpu.SemaphoreType.DMA((N_DEV - 1,)),
            pltpu.SemaphoreType.DMA((N_DEV - 1,)),
            pltpu.SemaphoreType.DMA((N_DEV - 1,)),
            pltpu.SemaphoreType.DMA((N_DEV - 1,)),
            pltpu.SemaphoreType.DMA((N_DEV - 1,)),
            pltpu.SemaphoreType.DMA((N_DEV - 1,)),
            pltpu.SemaphoreType.DMA((N_DEV - 1,)),
            pltpu.SemaphoreType.DMA((N_DEV - 1,)),
            pltpu.SemaphoreType.DMA((2,)),
            pltpu.SemaphoreType.REGULAR,
        ],
        compiler_params=pltpu.CompilerParams(
            collective_id=0, vmem_limit_bytes=100 * 1024 * 1024),
    )(x, w_mat, scale)


# device time: 321437 ns/iter; 1.0338x vs baseline; 1.0338x over previous
import jax
import jax.numpy as jnp
from jax import lax
from jax.experimental import pallas as pl
from jax.experimental.pallas import tpu as pltpu

N_DEV = 4
M = 4096
K_SHARD = 1024
N_TOT = 2048
MC = M // N_DEV
NH = N_TOT // 2


def kernel(x, w_mat, scale_x, scale_w):
    scale = (scale_x[0] * scale_w[0]).reshape(1, 1).astype(jnp.float32)

    def body(x_ref, w_ref, s_ref, out_ref, comm0, comm1, epi,
             rs_send0, rs_recv0, rs_send1, rs_recv1,
             ag_send0, ag_recv0, ag_send1, ag_recv1,
             epi_sems, credit_sem):
        i = lax.axis_index("i")
        right = lax.rem(i + 1, N_DEV)
        left = lax.rem(i + N_DEV - 1, N_DEV)
        tgt = (right, left)

        barrier = pltpu.get_barrier_semaphore()
        for nbr in (left, right):
            pl.semaphore_signal(barrier, inc=1, device_id=(nbr,),
                                device_id_type=pl.DeviceIdType.MESH)
        pl.semaphore_wait(barrier, 2)

        def mod4(v):
            return lax.rem(v + 2 * N_DEV, N_DEV)

        def pgemm(c, d):
            xa = x_ref[pl.ds(c * MC, MC), :].astype(jnp.bfloat16)
            wb = w_ref[:, d * NH:(d + 1) * NH].astype(jnp.bfloat16)
            return jnp.dot(xa, wb, preferred_element_type=jnp.float32)

        comms = (comm0, comm1)
        rs_send = (rs_send0, rs_send1)
        rs_recv = (rs_recv0, rs_recv1)

        def rs_rdma(h, d):
            return pltpu.make_async_remote_copy(
                src_ref=comms[d].at[h % 2],
                dst_ref=comms[d].at[(h + 1) % 2],
                send_sem=rs_send[d].at[h],
                recv_sem=rs_recv[d].at[h],
                device_id=(tgt[d],),
                device_id_type=pl.DeviceIdType.MESH,
            )

        comm0[0, :, :] = pgemm(mod4(i - 1), 0)
        r0 = rs_rdma(0, 0)
        r0.start()
        comm1[0, :, :] = pgemm(mod4(i + 1), 1)
        r1 = rs_rdma(0, 1)
        r1.start()
        pending = [r0, r1]

        for h in range(N_DEV - 1):
            recv_slot = (h + 1) % 2
            if h < N_DEV - 2:
                epi[0, :, :] = pgemm(mod4(i - 2 - h), 0)
                epi[1, :, :] = pgemm(mod4(i + 2 + h), 1)
                if h == 0:
                    for d in range(2):
                        pending[d].wait()
                        comms[d][recv_slot, :, :] = (
                            comms[d][recv_slot, :, :] + epi[d, :, :])
                        pending[d] = rs_rdma(h + 1, d)
                        pending[d].start()
                else:
                    for d in range(2):
                        pending[d].wait()
                        comms[d][recv_slot, :, :] = (
                            comms[d][recv_slot, :, :] + epi[d, :, :])
                    pl.semaphore_signal(credit_sem, inc=1, device_id=(left,),
                                        device_id_type=pl.DeviceIdType.MESH)
                    pl.semaphore_signal(credit_sem, inc=1, device_id=(right,),
                                        device_id_type=pl.DeviceIdType.MESH)
                    pl.semaphore_wait(credit_sem, 2)
                    for d in range(2):
                        pending[d] = rs_rdma(h + 1, d)
                        pending[d].start()
            else:
                epi[0, :, :] = pgemm(i, 0)
                epi[1, :, :] = pgemm(i, 1)
                s = s_ref[0, 0]
                copies = []
                for d in range(2):
                    pending[d].wait()
                    y = (comms[d][recv_slot, :, :] + epi[d, :, :]) * s
                    z = jnp.clip(y, -60.0, 60.0)
                    epi[d, :, :] = y / (1.0 + jnp.exp(-z))
                    cp = pltpu.make_async_copy(
                        epi.at[d],
                        out_ref.at[pl.ds(i * MC, MC), pl.ds(d * NH, NH)],
                        epi_sems.at[d],
                    )
                    cp.start()
                    copies.append(cp)

        ag_send = (ag_send0, ag_send1)
        ag_recv = (ag_recv0, ag_recv1)

        def ag_desc(h, d, c):
            region = out_ref.at[pl.ds(c * MC, MC), pl.ds(d * NH, NH)]
            return pltpu.make_async_remote_copy(
                src_ref=region,
                dst_ref=region,
                send_sem=ag_send[d].at[h],
                recv_sem=ag_recv[d].at[h],
                device_id=(tgt[d],),
                device_id_type=pl.DeviceIdType.MESH,
            )

        sends = []
        recvs = [None, None]
        for h in range(N_DEV - 1):
            send_c = (mod4(i - h), mod4(i + h))
            recv_c = (mod4(i - 1 - h), mod4(i + 1 + h))
            for d in range(2):
                if h == 0:
                    copies[d].wait()
                else:
                    recvs[d].wait_recv()
                send_desc = ag_desc(h, d, send_c[d])
                send_desc.start()
                sends.append(send_desc)
                recvs[d] = ag_desc(h, d, recv_c[d])
        for d in range(2):
            recvs[d].wait_recv()
        for send_desc in sends:
            send_desc.wait_send()

    return pl.pallas_call(
        body,
        out_shape=jax.ShapeDtypeStruct((M, N_TOT), jnp.float32),
        in_specs=[
            pl.BlockSpec(memory_space=pltpu.VMEM),
            pl.BlockSpec(memory_space=pltpu.VMEM),
            pl.BlockSpec(memory_space=pltpu.SMEM),
        ],
        out_specs=pl.BlockSpec(memory_space=pltpu.MemorySpace.HBM),
        scratch_shapes=[
            pltpu.VMEM((2, MC, NH), jnp.float32),
            pltpu.VMEM((2, MC, NH), jnp.float32),
            pltpu.VMEM((2, MC, NH), jnp.float32),
            pltpu.SemaphoreType.DMA((N_DEV - 1,)),
            pltpu.SemaphoreType.DMA((N_DEV - 1,)),
            pltpu.SemaphoreType.DMA((N_DEV - 1,)),
            pltpu.SemaphoreType.DMA((N_DEV - 1,)),
            pltpu.SemaphoreType.DMA((N_DEV - 1,)),
            pltpu.SemaphoreType.DMA((N_DEV - 1,)),
            pltpu.SemaphoreType.DMA((N_DEV - 1,)),
            pltpu.SemaphoreType.DMA((N_DEV - 1,)),
            pltpu.SemaphoreType.DMA((2,)),
            pltpu.SemaphoreType.REGULAR,
        ],
        compiler_params=pltpu.CompilerParams(
            collective_id=0, vmem_limit_bytes=100 * 1024 * 1024),
    )(x, w_mat, scale)


# device time: 311559 ns/iter; 1.0666x vs baseline; 1.0317x over previous
import jax
import jax.numpy as jnp
from jax import lax
from jax.experimental import pallas as pl
from jax.experimental.pallas import tpu as pltpu

N_DEV = 4
M = 4096
K_SHARD = 1024
N_TOT = 2048
MC = M // N_DEV
NH = N_TOT // 2
SUBS = 4
MCS = MC // SUBS


def kernel(x, w_mat, scale_x, scale_w):
    scale = (scale_x[0] * scale_w[0]).reshape(1, 1).astype(jnp.float32)

    def body(x_ref, w_ref, s_ref, out_ref, comm0, comm1, epi,
             rs_send0, rs_recv0, rs_send1, rs_recv1,
             ag_send0, ag_recv0, ag_send1, ag_recv1,
             epi_sems, credit_sem):
        i = lax.axis_index("i")
        right = lax.rem(i + 1, N_DEV)
        left = lax.rem(i + N_DEV - 1, N_DEV)
        tgt = (right, left)

        barrier = pltpu.get_barrier_semaphore()
        for nbr in (left, right):
            pl.semaphore_signal(barrier, inc=1, device_id=(nbr,),
                                device_id_type=pl.DeviceIdType.MESH)
        pl.semaphore_wait(barrier, 2)

        def mod4(v):
            return lax.rem(v + 2 * N_DEV, N_DEV)

        def pgemm(c, d):
            xa = x_ref[pl.ds(c * MC, MC), :].astype(jnp.bfloat16)
            wb = w_ref[:, d * NH:(d + 1) * NH].astype(jnp.bfloat16)
            return jnp.dot(xa, wb, preferred_element_type=jnp.float32)

        comms = (comm0, comm1)
        rs_send = (rs_send0, rs_send1)
        rs_recv = (rs_recv0, rs_recv1)
        ag_send = (ag_send0, ag_send1)
        ag_recv = (ag_recv0, ag_recv1)

        def rs_rdma(h, d, j):
            rows = pl.ds(j * MCS, MCS)
            return pltpu.make_async_remote_copy(
                src_ref=comms[d].at[h % 2, rows],
                dst_ref=comms[d].at[(h + 1) % 2, rows],
                send_sem=rs_send[d].at[h * SUBS + j],
                recv_sem=rs_recv[d].at[h * SUBS + j],
                device_id=(tgt[d],),
                device_id_type=pl.DeviceIdType.MESH,
            )

        rs_descs = {}
        comm0[0, :, :] = pgemm(mod4(i - 1), 0)
        for j in range(SUBS):
            r = rs_rdma(0, 0, j)
            r.start()
            rs_descs[(0, 0, j)] = r
        comm1[0, :, :] = pgemm(mod4(i + 1), 1)
        for j in range(SUBS):
            r = rs_rdma(0, 1, j)
            r.start()
            rs_descs[(0, 1, j)] = r

        epi[0, :, :] = pgemm(mod4(i - 2), 0)
        epi[1, :, :] = pgemm(mod4(i + 2), 1)
        for j in range(SUBS):
            sl = slice(j * MCS, (j + 1) * MCS)
            for d in range(2):
                rs_descs[(0, d, j)].wait_recv()
                comms[d][1, sl, :] = comms[d][1, sl, :] + epi[d, sl, :]
                r = rs_rdma(1, d, j)
                r.start()
                rs_descs[(1, d, j)] = r

        epi[0, :, :] = pgemm(mod4(i - 3), 0)
        epi[1, :, :] = pgemm(mod4(i + 3), 1)
        for j in range(SUBS):
            sl = slice(j * MCS, (j + 1) * MCS)
            for d in range(2):
                rs_descs[(1, d, j)].wait_recv()
                rs_descs[(0, d, j)].wait_send()
                comms[d][0, sl, :] = comms[d][0, sl, :] + epi[d, sl, :]
        for j in range(SUBS):
            for d in range(2):
                rs_descs[(1, d, j)].wait_send()
        pl.semaphore_signal(credit_sem, inc=1, device_id=(left,),
                            device_id_type=pl.DeviceIdType.MESH)
        pl.semaphore_signal(credit_sem, inc=1, device_id=(right,),
                            device_id_type=pl.DeviceIdType.MESH)
        pl.semaphore_wait(credit_sem, 2)
        for j in range(SUBS):
            for d in range(2):
                r = rs_rdma(2, d, j)
                r.start()
                rs_descs[(2, d, j)] = r

        epi[0, :, :] = pgemm(i, 0)
        epi[1, :, :] = pgemm(i, 1)
        s = s_ref[0, 0]
        copies = {}
        for j in range(SUBS):
            sl = slice(j * MCS, (j + 1) * MCS)
            for d in range(2):
                rs_descs[(2, d, j)].wait_recv()
                y = (comms[d][1, sl, :] + epi[d, sl, :]) * s
                z = jnp.clip(y, -60.0, 60.0)
                epi[d, sl, :] = y / (1.0 + jnp.exp(-z))
                cp = pltpu.make_async_copy(
                    epi.at[d, pl.ds(j * MCS, MCS)],
                    out_ref.at[pl.ds(i * MC + j * MCS, MCS),
                               pl.ds(d * NH, NH)],
                    epi_sems.at[d * SUBS + j],
                )
                cp.start()
                copies[(d, j)] = cp

        def ag_desc(h, d, j, c):
            region = out_ref.at[pl.ds(c * MC + j * MCS, MCS),
                                pl.ds(d * NH, NH)]
            return pltpu.make_async_remote_copy(
                src_ref=region,
                dst_ref=region,
                send_sem=ag_send[d].at[h * SUBS + j],
                recv_sem=ag_recv[d].at[h * SUBS + j],
                device_id=(tgt[d],),
                device_id_type=pl.DeviceIdType.MESH,
            )

        ag_sends = []
        ag_recvs = {}
        for h in range(N_DEV - 1):
            send_c = (mod4(i - h), mod4(i + h))
            recv_c = (mod4(i - 1 - h), mod4(i + 1 + h))
            for j in range(SUBS):
                for d in range(2):
                    if h == 0:
                        copies[(d, j)].wait()
                    else:
                        ag_recvs[(h - 1, d, j)].wait_recv()
                    sd = ag_desc(h, d, j, send_c[d])
                    sd.start()
                    ag_sends.append(sd)
                    ag_recvs[(h, d, j)] = ag_desc(h, d, j, recv_c[d])
        for j in range(SUBS):
            for d in range(2):
                ag_recvs[(N_DEV - 2, d, j)].wait_recv()
        for sd in ag_sends:
            sd.wait_send()
        for j in range(SUBS):
            for d in range(2):
                rs_descs[(2, d, j)].wait_send()

    return pl.pallas_call(
        body,
        out_shape=jax.ShapeDtypeStruct((M, N_TOT), jnp.float32),
        in_specs=[
            pl.BlockSpec(memory_space=pltpu.VMEM),
            pl.BlockSpec(memory_space=pltpu.VMEM),
            pl.BlockSpec(memory_space=pltpu.SMEM),
        ],
        out_specs=pl.BlockSpec(memory_space=pltpu.MemorySpace.HBM),
        scratch_shapes=[
            pltpu.VMEM((2, MC, NH), jnp.float32),
            pltpu.VMEM((2, MC, NH), jnp.float32),
            pltpu.VMEM((2, MC, NH), jnp.float32),
            pltpu.SemaphoreType.DMA(((N_DEV - 1) * SUBS,)),
            pltpu.SemaphoreType.DMA(((N_DEV - 1) * SUBS,)),
            pltpu.SemaphoreType.DMA(((N_DEV - 1) * SUBS,)),
            pltpu.SemaphoreType.DMA(((N_DEV - 1) * SUBS,)),
            pltpu.SemaphoreType.DMA(((N_DEV - 1) * SUBS,)),
            pltpu.SemaphoreType.DMA(((N_DEV - 1) * SUBS,)),
            pltpu.SemaphoreType.DMA(((N_DEV - 1) * SUBS,)),
            pltpu.SemaphoreType.DMA(((N_DEV - 1) * SUBS,)),
            pltpu.SemaphoreType.DMA((2 * SUBS,)),
            pltpu.SemaphoreType.REGULAR,
        ],
        compiler_params=pltpu.CompilerParams(
            collective_id=0, vmem_limit_bytes=100 * 1024 * 1024),
    )(x, w_mat, scale)
